# baseline (speedup 1.0000x reference)
import functools

import jax
import jax.numpy as jnp
from jax import lax
from jax.experimental import pallas as pl
from jax.experimental.pallas import tpu as pltpu
from jax.experimental.pallas import tpu_sc as plsc

N, E, NODE_DIM, EDGE_DIM, H, G = 10000, 320000, 128, 16, 64, 64
NC, NS = 2, 16
NW = NC * NS
EPW = E // NW
CH = 200
NCHUNK = EPW // CH
NP = 10240
IPW = 10240
RPT = NP // NS
LF32 = 16


def _edge_stage_body(x_hbm, e_hbm, src_hbm, dst_hbm, out_hbm,
                     src_v, dst_v, xg_a, xg_b, e_a, e_b, acc_sh,
                     gsem_a, gsem_b, esem_a, esem_b, ssem_a, ssem_b):
    cid = lax.axis_index("c")
    sid = lax.axis_index("s")
    wid = cid * NS + sid
    ebase = wid * EPW
    PIECE = RPT // 5

    def zero_row(i, carry):
        for k in range(H // LF32):
            xg_a[i, pl.ds(k * LF32, LF32)] = jnp.zeros((LF32,), jnp.float32)
        return carry
    lax.fori_loop(0, PIECE, zero_row, 0)
    for p in range(5):
        pltpu.sync_copy(xg_a.at[pl.ds(0, PIECE)],
                        acc_sh.at[pl.ds(sid * RPT + p * PIECE, PIECE)])

    pltpu.sync_copy(src_hbm.at[pl.ds(ebase, EPW)], src_v)
    pltpu.sync_copy(dst_hbm.at[pl.ds(ebase, EPW)], dst_v)
    plsc.subcore_barrier()

    def start_loads(j, xg_v, e_v, gsem, esem):
        pltpu.make_async_copy(x_hbm.at[src_v.at[pl.ds(j * CH, CH)]], xg_v,
                              gsem).start()
        pltpu.make_async_copy(e_hbm.at[pl.ds(ebase + j * CH, CH)], e_v,
                              esem).start()

    def wait_loads(xg_v, e_v, gsem, esem):
        pltpu.make_async_copy(x_hbm.at[src_v.at[pl.ds(0, CH)]], xg_v,
                              gsem).wait()
        pltpu.make_async_copy(e_hbm.at[pl.ds(ebase, CH)], e_v, esem).wait()

    def compute(xg_v, e_v):
        def row(i, c2):
            for k in range(H // LF32):
                s = pl.ds(k * LF32, LF32)
                xg_v[i, s] = jnp.maximum(xg_v[i, s] + e_v[i, s], 0.0)
            return c2
        lax.fori_loop(0, CH, row, 0)

    def start_scatter(j, xg_v, ssem):
        pltpu.async_copy(xg_v, acc_sh.at[dst_v.at[pl.ds(j * CH, CH)]], ssem,
                         add=True)

    def wait_scatter(xg_v, ssem):
        pltpu.make_async_copy(xg_v, acc_sh.at[dst_v.at[pl.ds(0, CH)]],
                              ssem).wait()

    bufs_a = (xg_a, e_a, gsem_a, esem_a)
    bufs_b = (xg_b, e_b, gsem_b, esem_b)

    start_loads(0, *bufs_a)

    def pair(j2, carry):
        a = 2 * j2
        b = a + 1
        @pl.when(j2 > 0)
        def _():
            wait_scatter(xg_b, ssem_b)
        start_loads(b, *bufs_b)
        wait_loads(*bufs_a)
        compute(xg_a, e_a)
        start_scatter(a, xg_a, ssem_a)
        wait_scatter(xg_a, ssem_a)

        @pl.when(b + 1 < NCHUNK)
        def _():
            start_loads(b + 1, *bufs_a)
        wait_loads(*bufs_b)
        compute(xg_b, e_b)
        start_scatter(b, xg_b, ssem_b)
        return carry
    lax.fori_loop(0, NCHUNK // 2, pair, 0)
    wait_scatter(xg_b, ssem_b)

    plsc.subcore_barrier()
    for p in range(5):
        rows = pl.ds(sid * RPT + p * PIECE, PIECE)
        pltpu.sync_copy(acc_sh.at[rows], xg_a.at[pl.ds(0, PIECE)])
        pltpu.sync_copy(xg_a.at[pl.ds(0, PIECE)], out_hbm.at[cid, rows])


_edge_stage = functools.partial(
    pl.kernel,
    mesh=plsc.VectorSubcoreMesh(core_axis_name="c", subcore_axis_name="s"),
    compiler_params=pltpu.CompilerParams(use_tc_tiling_on_sc=False),
    out_type=jax.ShapeDtypeStruct((NC, NP, H), jnp.float32),
    scratch_types=[
        pltpu.VMEM((EPW,), jnp.int32),
        pltpu.VMEM((EPW,), jnp.int32),
        pltpu.VMEM((CH, H), jnp.float32),
        pltpu.VMEM((CH, H), jnp.float32),
        pltpu.VMEM((CH, H), jnp.float32),
        pltpu.VMEM((CH, H), jnp.float32),
        pltpu.VMEM_SHARED((NP, H), jnp.float32),
        pltpu.SemaphoreType.DMA,
        pltpu.SemaphoreType.DMA,
        pltpu.SemaphoreType.DMA,
        pltpu.SemaphoreType.DMA,
        pltpu.SemaphoreType.DMA,
        pltpu.SemaphoreType.DMA,
    ],
)(_edge_stage_body)


def _node_enc_body(nf_ref, w_ref, b_ref, o_ref):
    o_ref[...] = jnp.maximum(
        jnp.dot(nf_ref[...], w_ref[...], preferred_element_type=jnp.float32)
        + b_ref[...], 0.0)


def _edge_enc_body(ef_ref, w_ref, b_ref, o_ref):
    o_ref[...] = jnp.dot(
        ef_ref[...], w_ref[...], preferred_element_type=jnp.float32) + b_ref[...]


def _mlp_body(x_ref, p_ref, wa_ref, ba_ref, wb_ref, bb_ref, o_ref):
    h = x_ref[...] + p_ref[0, :N] + p_ref[1, :N]
    t = jnp.maximum(
        jnp.dot(h, wa_ref[...], preferred_element_type=jnp.float32)
        + ba_ref[...], 0.0)
    o_ref[...] = jnp.maximum(
        jnp.dot(t, wb_ref[...], preferred_element_type=jnp.float32)
        + bb_ref[...], 0.0)


def _mlp_pool_head_body(x_ref, p_ref, wa_ref, ba_ref, wb_ref, bb_ref,
                        b1d_ref, w1_ref, b1_ref, w2_ref, b2_ref, o_ref):
    h = x_ref[...] + p_ref[0, :N] + p_ref[1, :N]
    t = jnp.maximum(
        jnp.dot(h, wa_ref[...], preferred_element_type=jnp.float32)
        + ba_ref[...], 0.0)
    x3 = jnp.maximum(
        jnp.dot(t, wb_ref[...], preferred_element_type=jnp.float32)
        + bb_ref[...], 0.0)
    oh_t = (lax.broadcasted_iota(jnp.int32, (G, 1), 0)
            == b1d_ref[...][None, :]).astype(jnp.float32)
    s = jnp.dot(oh_t, x3, preferred_element_type=jnp.float32)
    cnt = jnp.dot(oh_t, jnp.ones((N, 1), jnp.float32),
                  preferred_element_type=jnp.float32)
    pooled = s / jnp.maximum(cnt, 1.0)
    f = jnp.maximum(
        jnp.dot(pooled, w1_ref[...], preferred_element_type=jnp.float32)
        + b1_ref[...], 0.0)
    o_ref[...] = jnp.dot(
        f, w2_ref[...], preferred_element_type=jnp.float32) + b2_ref[...]


def _full(shape, dtype=jnp.float32):
    return jax.ShapeDtypeStruct(shape, dtype)


def kernel(node_features, edge_index, edge_features, batch,
           W_node, b_node, W_edge, b_edge,
           Wc0a, bc0a, Wc0b, bc0b,
           Wc1a, bc1a, Wc1b, bc1b,
           Wc2a, bc2a, Wc2b, bc2b,
           W_fc1, b_fc1, W_fc2, b_fc2):
    src = edge_index[0]
    dst = edge_index[1]

    x = pl.pallas_call(_node_enc_body, out_shape=_full((N, H)))(
        node_features, W_node, b_node)

    EB = 8000
    e = pl.pallas_call(
        _edge_enc_body,
        grid=(E // EB,),
        in_specs=[
            pl.BlockSpec((EB, EDGE_DIM), lambda i: (i, 0)),
            pl.BlockSpec((EDGE_DIM, H), lambda i: (0, 0)),
            pl.BlockSpec((H,), lambda i: (0,)),
        ],
        out_specs=pl.BlockSpec((EB, H), lambda i: (i, 0)),
        out_shape=_full((E, H)),
    )(edge_features, W_edge, b_edge)

    for Wa, ba, Wb, bb in [(Wc0a, bc0a, Wc0b, bc0b), (Wc1a, bc1a, Wc1b, bc1b)]:
        p = _edge_stage(x, e, src, dst)
        x = pl.pallas_call(_mlp_body, out_shape=_full((N, H)))(
            x, p, Wa, ba, Wb, bb)

    p = _edge_stage(x, e, src, dst)
    out = pl.pallas_call(_mlp_pool_head_body, out_shape=_full((G, 1)))(
        x, p, Wc2a, bc2a, Wc2b, bc2b, batch, W_fc1, b_fc1, W_fc2, b_fc2)
    return out


# speedup vs baseline: 1.1932x; 1.1932x over previous
import functools

import jax
import jax.numpy as jnp
from jax import lax
from jax.experimental import pallas as pl
from jax.experimental.pallas import tpu as pltpu
from jax.experimental.pallas import tpu_sc as plsc

N, E, NODE_DIM, EDGE_DIM, H, G = 10000, 320000, 128, 16, 64, 64
NC, NS = 2, 16
NW = NC * NS
EPW = E // NW
CH = 200
NCHUNK = EPW // CH
EP8 = E // 8
RPW8 = EPW // 8
CR = CH // 8
NP = 10240
IPW = 10240
RPT = NP // NS
LF32 = 16


def _edge_stage_body(x_hbm, e_hbm, src_hbm, dst_hbm, out_hbm,
                     src_v, dst_v, xg_a, xg_b, e_a, e_b, acc_sh,
                     gsem_a, gsem_b, esem_a, esem_b, ssem_a, ssem_b):
    cid = lax.axis_index("c")
    sid = lax.axis_index("s")
    wid = cid * NS + sid
    ebase = wid * EPW
    PIECE = RPT // 5

    def zero_row(i, carry):
        for k in range(H // LF32):
            xg_a[i, pl.ds(k * LF32, LF32)] = jnp.zeros((LF32,), jnp.float32)
        return carry
    lax.fori_loop(0, PIECE, zero_row, 0)
    for p in range(5):
        pltpu.sync_copy(xg_a.at[pl.ds(0, PIECE)],
                        acc_sh.at[pl.ds(sid * RPT + p * PIECE, PIECE)])

    pltpu.sync_copy(src_hbm.at[pl.ds(ebase, EPW)], src_v)
    pltpu.sync_copy(dst_hbm.at[pl.ds(ebase, EPW)], dst_v)
    plsc.subcore_barrier()

    rbase = wid * RPW8

    def start_loads(j, xg_v, e_v, gsem, esem):
        pltpu.make_async_copy(x_hbm.at[src_v.at[pl.ds(j * CH, CH)]], xg_v,
                              gsem).start()
        pltpu.make_async_copy(e_hbm.at[pl.ds(rbase + j * CR, CR)], e_v,
                              esem).start()

    def wait_loads(xg_v, e_v, gsem, esem):
        pltpu.make_async_copy(x_hbm.at[src_v.at[pl.ds(0, CH)]], xg_v,
                              gsem).wait()
        pltpu.make_async_copy(e_hbm.at[pl.ds(rbase, CR)], e_v, esem).wait()

    def compute(xg_v, e_v):
        def row(r, c2):
            b = 8 * r
            for m in range(8):
                for k in range(H // LF32):
                    s = pl.ds(k * LF32, LF32)
                    es = pl.ds(m * H + k * LF32, LF32)
                    xg_v[b + m, s] = jnp.maximum(
                        xg_v[b + m, s] + e_v[r, es], 0.0)
            return c2
        lax.fori_loop(0, CR, row, 0)

    def start_scatter(j, xg_v, ssem):
        pltpu.async_copy(xg_v, acc_sh.at[dst_v.at[pl.ds(j * CH, CH)]], ssem,
                         add=True)

    def wait_scatter(xg_v, ssem):
        pltpu.make_async_copy(xg_v, acc_sh.at[dst_v.at[pl.ds(0, CH)]],
                              ssem).wait()

    bufs_a = (xg_a, e_a, gsem_a, esem_a)
    bufs_b = (xg_b, e_b, gsem_b, esem_b)

    start_loads(0, *bufs_a)

    def pair(j2, carry):
        a = 2 * j2
        b = a + 1
        @pl.when(j2 > 0)
        def _():
            wait_scatter(xg_b, ssem_b)
        start_loads(b, *bufs_b)
        wait_loads(*bufs_a)
        compute(xg_a, e_a)
        start_scatter(a, xg_a, ssem_a)
        wait_scatter(xg_a, ssem_a)

        @pl.when(b + 1 < NCHUNK)
        def _():
            start_loads(b + 1, *bufs_a)
        wait_loads(*bufs_b)
        compute(xg_b, e_b)
        start_scatter(b, xg_b, ssem_b)
        return carry
    lax.fori_loop(0, NCHUNK // 2, pair, 0)
    wait_scatter(xg_b, ssem_b)

    plsc.subcore_barrier()
    for p in range(5):
        rows = pl.ds(sid * RPT + p * PIECE, PIECE)
        pltpu.sync_copy(acc_sh.at[rows], xg_a.at[pl.ds(0, PIECE)])
        pltpu.sync_copy(xg_a.at[pl.ds(0, PIECE)], out_hbm.at[cid, rows])


_edge_stage = functools.partial(
    pl.kernel,
    mesh=plsc.VectorSubcoreMesh(core_axis_name="c", subcore_axis_name="s"),
    compiler_params=pltpu.CompilerParams(use_tc_tiling_on_sc=False),
    out_type=jax.ShapeDtypeStruct((NC, NP, H), jnp.float32),
    scratch_types=[
        pltpu.VMEM((EPW,), jnp.int32),
        pltpu.VMEM((EPW,), jnp.int32),
        pltpu.VMEM((CH, H), jnp.float32),
        pltpu.VMEM((CH, H), jnp.float32),
        pltpu.VMEM((CR, 8 * H), jnp.float32),
        pltpu.VMEM((CR, 8 * H), jnp.float32),
        pltpu.VMEM_SHARED((NP, H), jnp.float32),
        pltpu.SemaphoreType.DMA,
        pltpu.SemaphoreType.DMA,
        pltpu.SemaphoreType.DMA,
        pltpu.SemaphoreType.DMA,
        pltpu.SemaphoreType.DMA,
        pltpu.SemaphoreType.DMA,
    ],
)(_edge_stage_body)


def _node_enc_body(nf_ref, w_ref, b_ref, o_ref):
    o_ref[...] = jnp.maximum(
        jnp.dot(nf_ref[...], w_ref[...], preferred_element_type=jnp.float32)
        + b_ref[...], 0.0)


def _edge_enc_body(ef_ref, w_ref, b_ref, o_ref):
    o_ref[...] = jnp.dot(
        ef_ref[...], w_ref[...], preferred_element_type=jnp.float32) + b_ref[...]


def _mlp_body(x_ref, p_ref, wa_ref, ba_ref, wb_ref, bb_ref, o_ref):
    h = x_ref[...] + p_ref[0, :N] + p_ref[1, :N]
    t = jnp.maximum(
        jnp.dot(h, wa_ref[...], preferred_element_type=jnp.float32)
        + ba_ref[...], 0.0)
    o_ref[...] = jnp.maximum(
        jnp.dot(t, wb_ref[...], preferred_element_type=jnp.float32)
        + bb_ref[...], 0.0)


def _mlp_pool_head_body(x_ref, p_ref, wa_ref, ba_ref, wb_ref, bb_ref,
                        b1d_ref, w1_ref, b1_ref, w2_ref, b2_ref, o_ref):
    h = x_ref[...] + p_ref[0, :N] + p_ref[1, :N]
    t = jnp.maximum(
        jnp.dot(h, wa_ref[...], preferred_element_type=jnp.float32)
        + ba_ref[...], 0.0)
    x3 = jnp.maximum(
        jnp.dot(t, wb_ref[...], preferred_element_type=jnp.float32)
        + bb_ref[...], 0.0)
    oh_t = (lax.broadcasted_iota(jnp.int32, (G, 1), 0)
            == b1d_ref[...][None, :]).astype(jnp.float32)
    s = jnp.dot(oh_t, x3, preferred_element_type=jnp.float32)
    cnt = jnp.dot(oh_t, jnp.ones((N, 1), jnp.float32),
                  preferred_element_type=jnp.float32)
    pooled = s / jnp.maximum(cnt, 1.0)
    f = jnp.maximum(
        jnp.dot(pooled, w1_ref[...], preferred_element_type=jnp.float32)
        + b1_ref[...], 0.0)
    o_ref[...] = jnp.dot(
        f, w2_ref[...], preferred_element_type=jnp.float32) + b2_ref[...]


def _full(shape, dtype=jnp.float32):
    return jax.ShapeDtypeStruct(shape, dtype)


def kernel(node_features, edge_index, edge_features, batch,
           W_node, b_node, W_edge, b_edge,
           Wc0a, bc0a, Wc0b, bc0b,
           Wc1a, bc1a, Wc1b, bc1b,
           Wc2a, bc2a, Wc2b, bc2b,
           W_fc1, b_fc1, W_fc2, b_fc2):
    src = edge_index[0]
    dst = edge_index[1]

    x = pl.pallas_call(_node_enc_body, out_shape=_full((N, H)))(
        node_features, W_node, b_node)

    ef8 = edge_features.reshape(EP8, 8 * EDGE_DIM)
    W8 = jnp.kron(jnp.eye(8, dtype=W_edge.dtype), W_edge)
    b8 = jnp.tile(b_edge, 8)
    EB = 2000
    e = pl.pallas_call(
        _edge_enc_body,
        grid=(EP8 // EB,),
        in_specs=[
            pl.BlockSpec((EB, 8 * EDGE_DIM), lambda i: (i, 0)),
            pl.BlockSpec((8 * EDGE_DIM, 8 * H), lambda i: (0, 0)),
            pl.BlockSpec((8 * H,), lambda i: (0,)),
        ],
        out_specs=pl.BlockSpec((EB, 8 * H), lambda i: (i, 0)),
        out_shape=_full((EP8, 8 * H)),
    )(ef8, W8, b8)

    for Wa, ba, Wb, bb in [(Wc0a, bc0a, Wc0b, bc0b), (Wc1a, bc1a, Wc1b, bc1b)]:
        p = _edge_stage(x, e, src, dst)
        x = pl.pallas_call(_mlp_body, out_shape=_full((N, H)))(
            x, p, Wa, ba, Wb, bb)

    p = _edge_stage(x, e, src, dst)
    out = pl.pallas_call(_mlp_pool_head_body, out_shape=_full((G, 1)))(
        x, p, Wc2a, bc2a, Wc2b, bc2b, batch, W_fc1, b_fc1, W_fc2, b_fc2)
    return out
